# baseline (speedup 1.0000x reference)
import jax
import jax.numpy as jnp
from jax.experimental import pallas as pl
from jax.experimental.pallas import tpu as pltpu

_K = 17
_PAD = (_K - 1) // 2
_TT = 256
_EXT = 2 * _PAD + 15


def _merge_pairs(n):
    pairs = []

    def merge(lo, m, r):
        step = r * 2
        if step < m:
            merge(lo, m, step)
            merge(lo + r, m, step)
            for i in range(lo + r, lo + m - r, step):
                pairs.append((i, i + r))
        else:
            pairs.append((lo, lo + r))

    merge(0, n, 1)
    return pairs


def _pruned_merge16():
    pairs = _merge_pairs(16)
    needed = {7, 8}
    kept = []
    for (a, b) in reversed(pairs):
        if a in needed or b in needed:
            kept.append((a, b, a in needed, b in needed))
            needed.add(a)
            needed.add(b)
    kept.reverse()
    return kept


_MERGE4 = [(a, b, True, True) for a, b in _merge_pairs(4)]
_MERGE8 = [(a, b, True, True) for a, b in _merge_pairs(8)]
_MERGE16_78 = _pruned_merge16()


def _apply(slots, ops):
    slots = list(slots)
    for a, b, need_min, need_max in ops:
        va, vb = slots[a], slots[b]
        if need_min:
            slots[a] = jnp.minimum(va, vb)
        if need_max:
            slots[b] = jnp.maximum(va, vb)
    return slots


def _sliding_median17(src, n_out):
    w1 = n_out + _EXT - 1
    s2 = _apply([src[0:w1], src[1:1 + w1]], [(0, 1, True, True)])
    w2 = w1 - 2
    s4 = _apply(
        [s2[0][0:w2], s2[1][0:w2], s2[0][2:2 + w2], s2[1][2:2 + w2]],
        _MERGE4,
    )
    w3 = w2 - 4
    m8 = _apply(
        [a[0:w3] for a in s4] + [a[4:4 + w3] for a in s4],
        _MERGE8,
    )
    w4 = w3 - 8
    m16 = _apply(
        [a[0:w4] for a in m8] + [a[8:8 + w4] for a in m8],
        _MERGE16_78,
    )
    r7 = m16[7][0:n_out]
    r8 = m16[8][0:n_out]
    x16 = src[2 * _PAD:2 * _PAD + n_out]
    return jnp.minimum(jnp.maximum(x16, r7), r8)


def _hpss_kernel(x_ref, oh_ref, op_ref, padT_ref):
    t = pl.program_id(1)
    col0 = pl.multiple_of(t * _TT, 128)
    f = oh_ref.shape[2]
    T = x_ref.shape[3]
    RT = padT_ref.shape[0]
    rows_f = f + _EXT + _PAD

    @pl.when(t == 0)
    def _():
        xb = x_ref[0, 0].astype(jnp.bfloat16)
        padT_ref[0:_PAD, :] = jnp.zeros((_PAD, f), jnp.bfloat16)
        padT_ref[_PAD + T:, :] = jnp.zeros((RT - _PAD - T, f), jnp.bfloat16)
        for j in range(T // _TT):
            padT_ref[_PAD + j * _TT:_PAD + (j + 1) * _TT, :] = jnp.transpose(
                xb[:, j * _TT:(j + 1) * _TT]
            )

    srcT = padT_ref[pl.ds(col0, _TT + _EXT), :]
    harm = jnp.transpose(_sliding_median17(srcT, _TT)).astype(jnp.float32)

    s = x_ref[0, 0, :, pl.ds(col0, _TT)]
    srcF = jnp.concatenate(
        [
            jnp.zeros((_PAD, _TT), jnp.bfloat16),
            s.astype(jnp.bfloat16),
            jnp.zeros((rows_f - _PAD - f, _TT), jnp.bfloat16),
        ],
        axis=0,
    )
    perc = _sliding_median17(srcF, f).astype(jnp.float32)

    z = jnp.maximum(harm, perc)
    tiny = jnp.finfo(jnp.float32).tiny
    z = jnp.where(z < tiny, jnp.float32(1.0), z)
    qh = harm / z
    qp = perc / z
    m = qh * qh
    r = qp * qp
    denom = m + r
    oh_ref[0, 0] = s * (m / denom)
    op_ref[0, 0] = s * (r / denom)


def kernel(S):
    B, C, F, T = S.shape
    nt = T // _TT
    rows_f = F + _EXT + _PAD
    rows_t = -(-(T + _EXT + _PAD) // 8) * 8
    outs = pl.pallas_call(
        _hpss_kernel,
        grid=(B * C, nt),
        in_specs=[
            pl.BlockSpec((1, 1, F, T), lambda b, t: (b // C, b % C, 0, 0))
        ],
        out_specs=[
            pl.BlockSpec((1, 1, F, _TT), lambda b, t: (b // C, b % C, 0, t)),
            pl.BlockSpec((1, 1, F, _TT), lambda b, t: (b // C, b % C, 0, t)),
        ],
        out_shape=[
            jax.ShapeDtypeStruct((B, C, F, T), S.dtype),
            jax.ShapeDtypeStruct((B, C, F, T), S.dtype),
        ],
        scratch_shapes=[
            pltpu.VMEM((rows_t, F), jnp.bfloat16),
        ],
        compiler_params=pltpu.CompilerParams(
            dimension_semantics=("parallel", "arbitrary"),
            vmem_limit_bytes=56 * 1024 * 1024,
        ),
        name="hpss_fused",
    )(S)
    return outs[0], outs[1]


# speedup vs baseline: 1.0091x; 1.0091x over previous
import jax
import jax.numpy as jnp
from jax.experimental import pallas as pl
from jax.experimental.pallas import tpu as pltpu

_K = 17
_PAD = (_K - 1) // 2
_TT = 256
_EXT = 2 * _PAD + 15


def _merge_pairs(n):
    pairs = []

    def merge(lo, m, r):
        step = r * 2
        if step < m:
            merge(lo, m, step)
            merge(lo + r, m, step)
            for i in range(lo + r, lo + m - r, step):
                pairs.append((i, i + r))
        else:
            pairs.append((lo, lo + r))

    merge(0, n, 1)
    return pairs


def _pruned_merge16():
    pairs = _merge_pairs(16)
    needed = {7, 8}
    kept = []
    for (a, b) in reversed(pairs):
        if a in needed or b in needed:
            kept.append((a, b, a in needed, b in needed))
            needed.add(a)
            needed.add(b)
    kept.reverse()
    return kept


_MERGE4 = [(a, b, True, True) for a, b in _merge_pairs(4)]
_MERGE8 = [(a, b, True, True) for a, b in _merge_pairs(8)]
_MERGE16_78 = _pruned_merge16()


def _apply(slots, ops):
    slots = list(slots)
    for a, b, need_min, need_max in ops:
        va, vb = slots[a], slots[b]
        if need_min:
            slots[a] = jnp.minimum(va, vb)
        if need_max:
            slots[b] = jnp.maximum(va, vb)
    return slots


def _sliding_median17(src, n_out):
    w1 = n_out + _EXT - 1
    s2 = _apply([src[0:w1], src[1:1 + w1]], [(0, 1, True, True)])
    w2 = w1 - 2
    s4 = _apply(
        [s2[0][0:w2], s2[1][0:w2], s2[0][2:2 + w2], s2[1][2:2 + w2]],
        _MERGE4,
    )
    w3 = w2 - 4
    m8 = _apply(
        [a[0:w3] for a in s4] + [a[4:4 + w3] for a in s4],
        _MERGE8,
    )
    w4 = w3 - 8
    m16 = _apply(
        [a[0:w4] for a in m8] + [a[8:8 + w4] for a in m8],
        _MERGE16_78,
    )
    r7 = m16[7][0:n_out]
    r8 = m16[8][0:n_out]
    x16 = src[2 * _PAD:2 * _PAD + n_out]
    return jnp.minimum(jnp.maximum(x16, r7), r8)


def _hpss_kernel(x_ref, oh_ref, op_ref, pad_ref, padT_ref):
    t = pl.program_id(1)
    col0 = pl.multiple_of(t * _TT, 128)
    f = oh_ref.shape[2]
    T = x_ref.shape[3]
    R = pad_ref.shape[0]
    RT = padT_ref.shape[0]

    @pl.when(t == 0)
    def _():
        xb = x_ref[0, 0].astype(jnp.bfloat16)
        pad_ref[0:_PAD, :] = jnp.zeros((_PAD, T), jnp.bfloat16)
        pad_ref[_PAD + f:, :] = jnp.zeros((R - _PAD - f, T), jnp.bfloat16)
        pad_ref[_PAD:_PAD + f, :] = xb
        padT_ref[0:_PAD, :] = jnp.zeros((_PAD, f), jnp.bfloat16)
        padT_ref[_PAD + T:, :] = jnp.zeros((RT - _PAD - T, f), jnp.bfloat16)
        for j in range(T // _TT):
            padT_ref[_PAD + j * _TT:_PAD + (j + 1) * _TT, :] = jnp.transpose(
                xb[:, j * _TT:(j + 1) * _TT]
            )

    srcT = padT_ref[pl.ds(col0, _TT + _EXT), :]
    harm = jnp.transpose(_sliding_median17(srcT, _TT)).astype(jnp.float32)

    srcF = pad_ref[:, pl.ds(col0, _TT)]
    perc = _sliding_median17(srcF, f).astype(jnp.float32)
    s = x_ref[0, 0, :, pl.ds(col0, _TT)]

    z = jnp.maximum(harm, perc)
    tiny = jnp.finfo(jnp.float32).tiny
    z = jnp.where(z < tiny, jnp.float32(1.0), z)
    qh = harm / z
    qp = perc / z
    m = qh * qh
    r = qp * qp
    denom = m + r
    oh_ref[0, 0] = s * (m / denom)
    op_ref[0, 0] = s * (r / denom)


def kernel(S):
    B, C, F, T = S.shape
    nt = T // _TT
    rows_f = F + _EXT + _PAD
    rows_t = -(-(T + _EXT + _PAD) // 8) * 8
    outs = pl.pallas_call(
        _hpss_kernel,
        grid=(B * C, nt),
        in_specs=[
            pl.BlockSpec((1, 1, F, T), lambda b, t: (b // C, b % C, 0, 0))
        ],
        out_specs=[
            pl.BlockSpec((1, 1, F, _TT), lambda b, t: (b // C, b % C, 0, t)),
            pl.BlockSpec((1, 1, F, _TT), lambda b, t: (b // C, b % C, 0, t)),
        ],
        out_shape=[
            jax.ShapeDtypeStruct((B, C, F, T), S.dtype),
            jax.ShapeDtypeStruct((B, C, F, T), S.dtype),
        ],
        scratch_shapes=[
            pltpu.VMEM((rows_f, T), jnp.bfloat16),
            pltpu.VMEM((rows_t, F), jnp.bfloat16),
        ],
        compiler_params=pltpu.CompilerParams(
            dimension_semantics=("parallel", "arbitrary"),
            vmem_limit_bytes=56 * 1024 * 1024,
        ),
        name="hpss_fused",
    )(S)
    return outs[0], outs[1]
